# baseline (speedup 1.0000x reference)
import jax
import jax.numpy as jnp
from jax.experimental import pallas as pl
from jax.experimental.pallas import tpu as pltpu
from jax.experimental.pallas import tpu_sc as plsc

BATCH = 16384
HIDDEN = 64
GATHER_WINDOW = 128


def _sc_gather(pp_paired, pn_paired, idx_p, idx_n):
    mesh = plsc.VectorSubcoreMesh(core_axis_name="core", subcore_axis_name="subcore")
    out_t = jax.ShapeDtypeStruct((BATCH, 2 * HIDDEN), jnp.float32)

    @pl.kernel(out_type=(out_t, out_t), mesh=mesh)
    def gather_kernel(pp_hbm, pn_hbm, ip_hbm, in_hbm, op_hbm, on_hbm):
        def body(ip_vmem, in_vmem, op_vmem, on_vmem):
            pltpu.sync_copy(pp_hbm.at[ip_vmem.at[0]], op_vmem)
            pltpu.sync_copy(pn_hbm.at[in_vmem.at[0]], on_vmem)

        pltpu.emit_pipeline(
            body,
            grid=(BATCH // GATHER_WINDOW,),
            in_specs=[
                pl.BlockSpec((1, GATHER_WINDOW), lambda i: (0, i)),
                pl.BlockSpec((1, GATHER_WINDOW), lambda i: (0, i)),
            ],
            out_specs=[
                pl.BlockSpec((GATHER_WINDOW, 2 * HIDDEN), lambda i: (i, 0)),
                pl.BlockSpec((GATHER_WINDOW, 2 * HIDDEN), lambda i: (i, 0)),
            ],
            core_axis_name=("core", "subcore"),
            dimension_semantics=(pltpu.PARALLEL,),
        )(ip_hbm, in_hbm, op_hbm, on_hbm)

    return gather_kernel(
        pp_paired,
        pn_paired,
        idx_p.reshape(1, BATCH),
        idx_n.reshape(1, BATCH),
    )


CHUNK = 1024
NCHUNK = BATCH // CHUNK


def _dot(a, b):
    return jax.lax.dot_general(
        a, b, (((1,), (0,)), ((), ())),
        precision=jax.lax.Precision.HIGHEST,
        preferred_element_type=jnp.float32)


def _mlp_body(p_ref, n_ref, x_ref, w1p_ref, w1n_ref, b1_ref, gamma_ref,
              beta_ref, w2_ref, b2_ref, w3_ref, b3_ref, out_ref,
              h1_ref, stat_ref):
    phase = pl.program_id(0)
    chunk = pl.program_id(1)

    @pl.when(phase == 0)
    def _pass1():
        @pl.when(chunk == 0)
        def _init():
            stat_ref[...] = jnp.zeros_like(stat_ref)

        xv = x_ref[...]
        par_p = (xv[:, 0:1] & 1) == 1
        par_n = (xv[:, 1:2] & 1) == 1
        p_full = p_ref[...]
        n_full = n_ref[...]
        p = jnp.where(par_p, p_full[:, HIDDEN:], p_full[:, :HIDDEN])
        n = jnp.where(par_n, n_full[:, HIDDEN:], n_full[:, :HIDDEN])
        p_norm = jnp.sqrt(jnp.sum(p * p, axis=-1, keepdims=True))
        n_norm = jnp.sqrt(jnp.sum(n * n, axis=-1, keepdims=True))
        p = p / jnp.maximum(p_norm, 1e-12)
        n = n / jnp.maximum(n_norm, 1e-12)
        h = _dot(p, w1p_ref[...]) + _dot(n, w1n_ref[...]) + b1_ref[...]
        h = jnp.maximum(h, 0.0)
        h1_ref[pl.ds(chunk * CHUNK, CHUNK), :] = h
        stat_ref[0:1, :] += jnp.sum(h, axis=0, keepdims=True)
        stat_ref[1:2, :] += jnp.sum(h * h, axis=0, keepdims=True)

    @pl.when(phase == 1)
    def _pass2():
        inv_n = 1.0 / BATCH
        mean = stat_ref[0:1, :] * inv_n
        var = stat_ref[1:2, :] * inv_n - mean * mean
        h = h1_ref[pl.ds(chunk * CHUNK, CHUNK), :]
        h = (h - mean) / jnp.sqrt(var + 1e-5) * gamma_ref[...] + beta_ref[...]
        h = jnp.maximum(_dot(h, w2_ref[...]) + b2_ref[...], 0.0)
        out_ref[...] = _dot(h, w3_ref[...]) + b3_ref[...]


def _mlp(p_rows, n_rows, x, W1, b1, gamma, beta, W2, b2, W3, b3, *,
         interpret=False):
    n_obs = W3.shape[0]

    def chunk_map(ph, c):
        return (jnp.where(ph == 0, c, 0), 0)

    def bcast_map(ph, c):
        return (0, 0)

    return pl.pallas_call(
        _mlp_body,
        grid=(2, NCHUNK),
        in_specs=[
            pl.BlockSpec((CHUNK, 2 * HIDDEN), chunk_map),
            pl.BlockSpec((CHUNK, 2 * HIDDEN), chunk_map),
            pl.BlockSpec((CHUNK, 2), chunk_map),
            pl.BlockSpec((HIDDEN, HIDDEN), bcast_map),
            pl.BlockSpec((HIDDEN, HIDDEN), bcast_map),
            pl.BlockSpec((1, HIDDEN), bcast_map),
            pl.BlockSpec((1, HIDDEN), bcast_map),
            pl.BlockSpec((1, HIDDEN), bcast_map),
            pl.BlockSpec((HIDDEN, HIDDEN), bcast_map),
            pl.BlockSpec((1, HIDDEN), bcast_map),
            pl.BlockSpec((HIDDEN, n_obs), bcast_map),
            pl.BlockSpec((1, n_obs), bcast_map),
        ],
        out_specs=pl.BlockSpec((CHUNK, n_obs),
                               lambda ph, c: (jnp.where(ph == 1, c, 0), 0)),
        out_shape=jax.ShapeDtypeStruct((BATCH, n_obs), jnp.float32),
        scratch_shapes=[
            pltpu.VMEM((BATCH, HIDDEN), jnp.float32),
            pltpu.VMEM((2, HIDDEN), jnp.float32),
        ],
        interpret=interpret,
    )(
        p_rows,
        n_rows,
        x,
        W1[:, :HIDDEN].T,
        W1[:, HIDDEN:].T,
        b1.reshape(1, -1),
        gamma.reshape(1, -1),
        beta.reshape(1, -1),
        W2.T,
        b2.reshape(1, -1),
        W3.T,
        b3.reshape(1, -1),
    )


def kernel(x, pos_proton, pos_neutron, W1, b1, gamma, beta, W2, b2, W3, b3):
    pp_paired = pos_proton.reshape(-1, 2 * HIDDEN)
    pn_paired = pos_neutron.reshape(-1, 2 * HIDDEN)
    idx_p = x[:, 0] >> 1
    idx_n = x[:, 1] >> 1
    p_rows, n_rows = _sc_gather(pp_paired, pn_paired, idx_p, idx_n)
    return _mlp(p_rows, n_rows, x, W1, b1, gamma, beta, W2, b2, W3, b3)


# speedup vs baseline: 1.5666x; 1.5666x over previous
import jax
import jax.numpy as jnp
from jax.experimental import pallas as pl
from jax.experimental.pallas import tpu as pltpu
from jax.experimental.pallas import tpu_sc as plsc

BATCH = 16384
HIDDEN = 64
GATHER_WINDOW = 256
CHUNK = 4096
NCHUNK = BATCH // CHUNK


def _sc_gather(pos_proton, pos_neutron, idx_p, idx_n):
    mesh = plsc.VectorSubcoreMesh(core_axis_name="core", subcore_axis_name="subcore")
    out_t = jax.ShapeDtypeStruct((BATCH, HIDDEN), jnp.float32)

    @pl.kernel(out_type=(out_t, out_t), mesh=mesh,
               compiler_params=pltpu.CompilerParams(use_tc_tiling_on_sc=False))
    def gather_kernel(pp_hbm, pn_hbm, ip_hbm, in_hbm, op_hbm, on_hbm):
        def body(ip_vmem, in_vmem, op_vmem, on_vmem):
            pltpu.sync_copy(pp_hbm.at[ip_vmem.at[0]], op_vmem)
            pltpu.sync_copy(pn_hbm.at[in_vmem.at[0]], on_vmem)

        pltpu.emit_pipeline(
            body,
            grid=(BATCH // GATHER_WINDOW,),
            in_specs=[
                pl.BlockSpec((1, GATHER_WINDOW), lambda i: (0, i)),
                pl.BlockSpec((1, GATHER_WINDOW), lambda i: (0, i)),
            ],
            out_specs=[
                pl.BlockSpec((GATHER_WINDOW, HIDDEN), lambda i: (i, 0)),
                pl.BlockSpec((GATHER_WINDOW, HIDDEN), lambda i: (i, 0)),
            ],
            core_axis_name=("core", "subcore"),
            dimension_semantics=(pltpu.PARALLEL,),
        )(ip_hbm, in_hbm, op_hbm, on_hbm)

    return gather_kernel(
        pos_proton,
        pos_neutron,
        idx_p.reshape(1, BATCH),
        idx_n.reshape(1, BATCH),
    )


def _dot(a, b):
    return jax.lax.dot_general(
        a, b, (((1,), (0,)), ((), ())), preferred_element_type=jnp.float32)


def _mlp_body(p_ref, n_ref, w1p_ref, w1n_ref, b1_ref, gamma_ref,
              beta_ref, w2_ref, b2_ref, w3_ref, b3_ref, out_ref,
              h1_ref, stat_ref):
    phase = pl.program_id(0)
    chunk = pl.program_id(1)

    @pl.when(phase == 0)
    def _pass1():
        @pl.when(chunk == 0)
        def _init():
            stat_ref[...] = jnp.zeros_like(stat_ref)

        p = p_ref[...]
        n = n_ref[...]
        p_norm = jnp.sqrt(jnp.sum(p * p, axis=-1, keepdims=True))
        n_norm = jnp.sqrt(jnp.sum(n * n, axis=-1, keepdims=True))
        p = p / jnp.maximum(p_norm, 1e-12)
        n = n / jnp.maximum(n_norm, 1e-12)
        h = _dot(p, w1p_ref[...]) + _dot(n, w1n_ref[...]) + b1_ref[...]
        h = jnp.maximum(h, 0.0)
        h1_ref[pl.ds(chunk * CHUNK, CHUNK), :] = h
        stat_ref[0:1, :] += jnp.sum(h, axis=0, keepdims=True)
        stat_ref[1:2, :] += jnp.sum(h * h, axis=0, keepdims=True)

    @pl.when(phase == 1)
    def _pass2():
        inv_n = 1.0 / BATCH
        mean = stat_ref[0:1, :] * inv_n
        var = stat_ref[1:2, :] * inv_n - mean * mean
        h = h1_ref[pl.ds(chunk * CHUNK, CHUNK), :]
        h = (h - mean) / jnp.sqrt(var + 1e-5) * gamma_ref[...] + beta_ref[...]
        h = jnp.maximum(_dot(h, w2_ref[...]) + b2_ref[...], 0.0)
        out_ref[...] = _dot(h, w3_ref[...]) + b3_ref[...]


def _mlp(p_rows, n_rows, W1, b1, gamma, beta, W2, b2, W3, b3, *,
         interpret=False):
    n_obs = W3.shape[0]

    def chunk_map(ph, c):
        return (jnp.where(ph == 0, c, 0), 0)

    def bcast_map(ph, c):
        return (0, 0)

    return pl.pallas_call(
        _mlp_body,
        grid=(2, NCHUNK),
        in_specs=[
            pl.BlockSpec((CHUNK, HIDDEN), chunk_map),
            pl.BlockSpec((CHUNK, HIDDEN), chunk_map),
            pl.BlockSpec((HIDDEN, HIDDEN), bcast_map),
            pl.BlockSpec((HIDDEN, HIDDEN), bcast_map),
            pl.BlockSpec((1, HIDDEN), bcast_map),
            pl.BlockSpec((1, HIDDEN), bcast_map),
            pl.BlockSpec((1, HIDDEN), bcast_map),
            pl.BlockSpec((HIDDEN, HIDDEN), bcast_map),
            pl.BlockSpec((1, HIDDEN), bcast_map),
            pl.BlockSpec((HIDDEN, n_obs), bcast_map),
            pl.BlockSpec((1, n_obs), bcast_map),
        ],
        out_specs=pl.BlockSpec((CHUNK, n_obs),
                               lambda ph, c: (jnp.where(ph == 1, c, 0), 0)),
        out_shape=jax.ShapeDtypeStruct((BATCH, n_obs), jnp.float32),
        scratch_shapes=[
            pltpu.VMEM((BATCH, HIDDEN), jnp.float32),
            pltpu.VMEM((2, HIDDEN), jnp.float32),
        ],
        interpret=interpret,
    )(
        p_rows,
        n_rows,
        W1[:, :HIDDEN].T,
        W1[:, HIDDEN:].T,
        b1.reshape(1, -1),
        gamma.reshape(1, -1),
        beta.reshape(1, -1),
        W2.T,
        b2.reshape(1, -1),
        W3.T,
        b3.reshape(1, -1),
    )


def kernel(x, pos_proton, pos_neutron, W1, b1, gamma, beta, W2, b2, W3, b3):
    idx_p = x[:, 0]
    idx_n = x[:, 1]
    p_rows, n_rows = _sc_gather(pos_proton, pos_neutron, idx_p, idx_n)
    return _mlp(p_rows, n_rows, W1, b1, gamma, beta, W2, b2, W3, b3)
